# baseline (speedup 1.0000x reference)
import jax
import jax.numpy as jnp
from jax.experimental import pallas as pl
from jax.experimental.pallas import tpu as pltpu

CHUNK = 1024


def _rpq_kernel(x_ref, p_ref, baug_ref, out_ref):
    x = x_ref[0]
    t = jnp.dot(x, p_ref[...], preferred_element_type=jnp.float32,
                precision=jax.lax.Precision.HIGHEST)
    n = jnp.sqrt(jnp.sum(t * t, axis=-1, keepdims=True))
    nn = jnp.maximum(n, 1e-12)
    L = x.shape[0]
    K = baug_ref.shape[1]
    a = jnp.concatenate(
        [t, nn, jnp.zeros((L, 7), jnp.float32)], axis=1)

    iota = jax.lax.broadcasted_iota(jnp.int32, (L, CHUNK), 1)
    best_val = jnp.full((L, 1), jnp.inf, jnp.float32)
    best_idx = jnp.zeros((L, 1), jnp.int32)
    for c in range(K // CHUNK):
        s = jnp.dot(a, baug_ref[:, c * CHUNK:(c + 1) * CHUNK],
                    preferred_element_type=jnp.float32,
                    precision=jax.lax.Precision.HIGHEST)
        cmin = jnp.min(s, axis=1, keepdims=True)
        carg = jnp.min(jnp.where(s == cmin, iota + c * CHUNK, K),
                       axis=1, keepdims=True)
        upd = cmin < best_val
        best_val = jnp.where(upd, cmin, best_val)
        best_idx = jnp.where(upd, carg, best_idx)
    out_ref[0] = best_idx


@jax.jit
def kernel(masked_target_values, project_mat, codebook_norm):
    B, L, D = masked_target_values.shape
    K, C = codebook_norm.shape
    csq = jnp.sum(codebook_norm * codebook_norm, axis=-1)
    baug = jnp.concatenate(
        [-2.0 * codebook_norm.T, csq[None, :],
         jnp.zeros((7, K), jnp.float32)], axis=0)

    out = pl.pallas_call(
        _rpq_kernel,
        grid=(B,),
        in_specs=[
            pl.BlockSpec((1, L, D), lambda b: (b, 0, 0)),
            pl.BlockSpec((D, C), lambda b: (0, 0)),
            pl.BlockSpec((24, K), lambda b: (0, 0)),
        ],
        out_specs=pl.BlockSpec((1, L, 1), lambda b: (b, 0, 0)),
        out_shape=jax.ShapeDtypeStruct((B, L, 1), jnp.int32),
    )(masked_target_values, project_mat, baug)
    return out[:, :, 0]


# speedup vs baseline: 3.9819x; 3.9819x over previous
import jax
import jax.numpy as jnp
from jax.experimental import pallas as pl
from jax.experimental.pallas import tpu as pltpu

CHUNK = 1024


def _rpq_kernel(x_ref, p_ref, ct_ref, out_ref):
    x = x_ref[0]
    t = jnp.dot(x, p_ref[...], preferred_element_type=jnp.float32)
    L = x.shape[0]
    K = ct_ref.shape[1]

    iota = jax.lax.broadcasted_iota(jnp.int32, (L, CHUNK), 1)
    best_val = jnp.full((L, 1), -jnp.inf, jnp.float32)
    best_idx = jnp.zeros((L, 1), jnp.int32)
    for c in range(K // CHUNK):
        s = jnp.dot(t, ct_ref[:, c * CHUNK:(c + 1) * CHUNK],
                    preferred_element_type=jnp.float32)
        cmax = jnp.max(s, axis=1, keepdims=True)
        carg = jnp.min(jnp.where(s == cmax, iota + c * CHUNK, K),
                       axis=1, keepdims=True)
        upd = cmax > best_val
        best_val = jnp.where(upd, cmax, best_val)
        best_idx = jnp.where(upd, carg, best_idx)
    out_ref[0] = best_idx


@jax.jit
def kernel(masked_target_values, project_mat, codebook_norm):
    B, L, D = masked_target_values.shape
    K, C = codebook_norm.shape
    ct = codebook_norm.T

    out = pl.pallas_call(
        _rpq_kernel,
        grid=(B,),
        in_specs=[
            pl.BlockSpec((1, L, D), lambda b: (b, 0, 0)),
            pl.BlockSpec((D, C), lambda b: (0, 0)),
            pl.BlockSpec((C, K), lambda b: (0, 0)),
        ],
        out_specs=pl.BlockSpec((1, L, 1), lambda b: (b, 0, 0)),
        out_shape=jax.ShapeDtypeStruct((B, L, 1), jnp.int32),
    )(masked_target_values, project_mat, ct)
    return out[:, :, 0]


# speedup vs baseline: 3.9837x; 1.0005x over previous
import jax
import jax.numpy as jnp
from jax.experimental import pallas as pl
from jax.experimental.pallas import tpu as pltpu

CHUNK = 1024


def _rpq_kernel(x_ref, p_ref, ct_ref, out_ref):
    x = x_ref[0]
    t = jnp.dot(x, p_ref[...], preferred_element_type=jnp.float32)
    nrm = jnp.sqrt(jnp.sum(t * t, axis=-1, keepdims=True))
    tn = t / jnp.maximum(nrm, 1e-12)
    L = x.shape[0]
    K = ct_ref.shape[1]

    iota = jax.lax.broadcasted_iota(jnp.int32, (L, CHUNK), 1)
    best_val = jnp.full((L, 1), -jnp.inf, jnp.float32)
    best_idx = jnp.zeros((L, 1), jnp.int32)
    for c in range(K // CHUNK):
        s = jnp.dot(tn, ct_ref[:, c * CHUNK:(c + 1) * CHUNK],
                    preferred_element_type=jnp.float32)
        cmax = jnp.max(s, axis=1, keepdims=True)
        carg = jnp.min(jnp.where(s == cmax, iota + c * CHUNK, K),
                       axis=1, keepdims=True)
        upd = cmax > best_val
        best_val = jnp.where(upd, cmax, best_val)
        best_idx = jnp.where(upd, carg, best_idx)
    out_ref[0] = best_idx


@jax.jit
def kernel(masked_target_values, project_mat, codebook_norm):
    B, L, D = masked_target_values.shape
    K, C = codebook_norm.shape
    ct = codebook_norm.T

    out = pl.pallas_call(
        _rpq_kernel,
        grid=(B,),
        in_specs=[
            pl.BlockSpec((1, L, D), lambda b: (b, 0, 0)),
            pl.BlockSpec((D, C), lambda b: (0, 0)),
            pl.BlockSpec((C, K), lambda b: (0, 0)),
        ],
        out_specs=pl.BlockSpec((1, L, 1), lambda b: (b, 0, 0)),
        out_shape=jax.ShapeDtypeStruct((B, L, 1), jnp.int32),
    )(masked_target_values, project_mat, ct)
    return out[:, :, 0]


# speedup vs baseline: 5.2448x; 1.3166x over previous
import jax
import jax.numpy as jnp
from jax.experimental import pallas as pl
from jax.experimental.pallas import tpu as pltpu

CHUNK = 1024
SLAB = 128


def _rpq_kernel(x_ref, p_ref, ct_ref, out_ref):
    x = x_ref[0]
    t = jnp.dot(x, p_ref[...], preferred_element_type=jnp.float32)
    nrm = jnp.sqrt(jnp.sum(t * t, axis=-1, keepdims=True))
    tn = t / jnp.maximum(nrm, 1e-12)
    L = x.shape[0]
    K = ct_ref.shape[1]

    lane_f = jax.lax.broadcasted_iota(
        jnp.int32, (L, SLAB), 1).astype(jnp.float32)
    best_val = jnp.full((L, 1), -jnp.inf, jnp.float32)
    best_idx = jnp.zeros((L, 1), jnp.float32)
    for c in range(K // CHUNK):
        s = jnp.dot(tn, ct_ref[:, c * CHUNK:(c + 1) * CHUNK],
                    preferred_element_type=jnp.float32)
        bv = s[:, 0:SLAB]
        bg = jnp.zeros((L, SLAB), jnp.float32)
        for g in range(1, CHUNK // SLAB):
            sg = s[:, g * SLAB:(g + 1) * SLAB]
            m = sg > bv
            bv = jnp.maximum(bv, sg)
            bg = jnp.where(m, jnp.float32(g), bg)
        cmax = jnp.max(bv, axis=1, keepdims=True)
        gidx = bg * SLAB + lane_f
        cand = jnp.where(bv == cmax, gidx, jnp.float32(K))
        carg = jnp.min(cand, axis=1, keepdims=True) + c * CHUNK
        upd = cmax > best_val
        best_val = jnp.where(upd, cmax, best_val)
        best_idx = jnp.where(upd, carg, best_idx)
    out_ref[0] = best_idx.astype(jnp.int32)


@jax.jit
def kernel(masked_target_values, project_mat, codebook_norm):
    B, L, D = masked_target_values.shape
    K, C = codebook_norm.shape
    ct = codebook_norm.T

    out = pl.pallas_call(
        _rpq_kernel,
        grid=(B,),
        in_specs=[
            pl.BlockSpec((1, L, D), lambda b: (b, 0, 0)),
            pl.BlockSpec((D, C), lambda b: (0, 0)),
            pl.BlockSpec((C, K), lambda b: (0, 0)),
        ],
        out_specs=pl.BlockSpec((1, L, 1), lambda b: (b, 0, 0)),
        out_shape=jax.ShapeDtypeStruct((B, L, 1), jnp.int32),
    )(masked_target_values, project_mat, ct)
    return out[:, :, 0]


# speedup vs baseline: 5.9129x; 1.1274x over previous
import jax
import jax.numpy as jnp
from jax.experimental import pallas as pl
from jax.experimental.pallas import tpu as pltpu

CHUNK = 2048
SLAB = 128


def _rpq_kernel(x_ref, p_ref, ct_ref, out_ref):
    x = x_ref[0]
    t = jnp.dot(x, p_ref[...], preferred_element_type=jnp.float32)
    nrm = jnp.sqrt(jnp.sum(t * t, axis=-1, keepdims=True))
    tn = t / jnp.maximum(nrm, 1e-12)
    L = x.shape[0]
    K = ct_ref.shape[1]

    lane_f = jax.lax.broadcasted_iota(
        jnp.int32, (L, SLAB), 1).astype(jnp.float32)
    bv = jnp.full((L, SLAB), -jnp.inf, jnp.float32)
    bg = jnp.zeros((L, SLAB), jnp.float32)
    for c in range(K // CHUNK):
        s = jnp.dot(tn, ct_ref[:, c * CHUNK:(c + 1) * CHUNK],
                    preferred_element_type=jnp.float32)
        for g in range(CHUNK // SLAB):
            gg = c * (CHUNK // SLAB) + g
            sg = s[:, g * SLAB:(g + 1) * SLAB]
            if gg == 0:
                bv = sg
            else:
                m = sg > bv
                bv = jnp.maximum(bv, sg)
                bg = jnp.where(m, jnp.float32(gg), bg)
    cmax = jnp.max(bv, axis=1, keepdims=True)
    gidx = bg * SLAB + lane_f
    cand = jnp.where(bv == cmax, gidx, jnp.float32(K))
    best_idx = jnp.min(cand, axis=1, keepdims=True)
    out_ref[0] = best_idx.astype(jnp.int32)


@jax.jit
def kernel(masked_target_values, project_mat, codebook_norm):
    B, L, D = masked_target_values.shape
    K, C = codebook_norm.shape
    ct = codebook_norm.T

    out = pl.pallas_call(
        _rpq_kernel,
        grid=(B,),
        in_specs=[
            pl.BlockSpec((1, L, D), lambda b: (b, 0, 0)),
            pl.BlockSpec((D, C), lambda b: (0, 0)),
            pl.BlockSpec((C, K), lambda b: (0, 0)),
        ],
        out_specs=pl.BlockSpec((1, L, 1), lambda b: (b, 0, 0)),
        out_shape=jax.ShapeDtypeStruct((B, L, 1), jnp.int32),
    )(masked_target_values, project_mat, ct)
    return out[:, :, 0]


# speedup vs baseline: 5.9160x; 1.0005x over previous
import jax
import jax.numpy as jnp
from jax.experimental import pallas as pl
from jax.experimental.pallas import tpu as pltpu

CHUNK = 4096
SLAB = 128


def _rpq_kernel(x_ref, p_ref, ct_ref, out_ref):
    x = x_ref[0]
    t = jnp.dot(x, p_ref[...], preferred_element_type=jnp.float32)
    nrm = jnp.sqrt(jnp.sum(t * t, axis=-1, keepdims=True))
    tn = t / jnp.maximum(nrm, 1e-12)
    L = x.shape[0]
    K = ct_ref.shape[1]

    lane_f = jax.lax.broadcasted_iota(
        jnp.int32, (L, SLAB), 1).astype(jnp.float32)
    bv = jnp.full((L, SLAB), -jnp.inf, jnp.float32)
    bg = jnp.zeros((L, SLAB), jnp.float32)
    for c in range(K // CHUNK):
        s = jnp.dot(tn, ct_ref[:, c * CHUNK:(c + 1) * CHUNK],
                    preferred_element_type=jnp.float32)
        for g in range(CHUNK // SLAB):
            gg = c * (CHUNK // SLAB) + g
            sg = s[:, g * SLAB:(g + 1) * SLAB]
            if gg == 0:
                bv = sg
            else:
                m = sg > bv
                bv = jnp.maximum(bv, sg)
                bg = jnp.where(m, jnp.float32(gg), bg)
    cmax = jnp.max(bv, axis=1, keepdims=True)
    gidx = bg * SLAB + lane_f
    cand = jnp.where(bv == cmax, gidx, jnp.float32(K))
    best_idx = jnp.min(cand, axis=1, keepdims=True)
    out_ref[0] = best_idx.astype(jnp.int32)


@jax.jit
def kernel(masked_target_values, project_mat, codebook_norm):
    B, L, D = masked_target_values.shape
    K, C = codebook_norm.shape
    ct = codebook_norm.T

    out = pl.pallas_call(
        _rpq_kernel,
        grid=(B,),
        in_specs=[
            pl.BlockSpec((1, L, D), lambda b: (b, 0, 0)),
            pl.BlockSpec((D, C), lambda b: (0, 0)),
            pl.BlockSpec((C, K), lambda b: (0, 0)),
        ],
        out_specs=pl.BlockSpec((1, L, 1), lambda b: (b, 0, 0)),
        out_shape=jax.ShapeDtypeStruct((B, L, 1), jnp.int32),
    )(masked_target_values, project_mat, ct)
    return out[:, :, 0]


# speedup vs baseline: 5.9269x; 1.0018x over previous
import jax
import jax.numpy as jnp
from jax.experimental import pallas as pl
from jax.experimental.pallas import tpu as pltpu

CHUNK = 2048
SLAB = 128


def _rpq_kernel(x_ref, p_ref, ct_ref, out_ref):
    x = x_ref[0]
    t = jnp.dot(x, p_ref[...], preferred_element_type=jnp.float32)
    nrm = jnp.sqrt(jnp.sum(t * t, axis=-1, keepdims=True))
    tn = t / jnp.maximum(nrm, 1e-12)
    L = x.shape[0]
    K = ct_ref.shape[1]

    lane_f = jax.lax.broadcasted_iota(
        jnp.int32, (L, SLAB), 1).astype(jnp.float32)
    bv = jnp.full((L, SLAB), -jnp.inf, jnp.float32)
    bg = jnp.zeros((L, SLAB), jnp.float32)
    for c in range(K // CHUNK):
        s = jnp.dot(tn, ct_ref[:, c * CHUNK:(c + 1) * CHUNK],
                    preferred_element_type=jnp.float32)
        for g in range(CHUNK // SLAB):
            gg = c * (CHUNK // SLAB) + g
            sg = s[:, g * SLAB:(g + 1) * SLAB]
            if gg == 0:
                bv = sg
            else:
                m = sg > bv
                bv = jnp.maximum(bv, sg)
                bg = jnp.where(m, jnp.float32(gg), bg)
    cmax = jnp.max(bv, axis=1, keepdims=True)
    gidx = bg * SLAB + lane_f
    cand = jnp.where(bv == cmax, gidx, jnp.float32(K))
    best_idx = jnp.min(cand, axis=1, keepdims=True)
    out_ref[0] = best_idx.astype(jnp.int32)


@jax.jit
def kernel(masked_target_values, project_mat, codebook_norm):
    B, L, D = masked_target_values.shape
    K, C = codebook_norm.shape
    ct = codebook_norm.T

    out = pl.pallas_call(
        _rpq_kernel,
        grid=(B,),
        in_specs=[
            pl.BlockSpec((1, L, D), lambda b: (b, 0, 0)),
            pl.BlockSpec((D, C), lambda b: (0, 0)),
            pl.BlockSpec((C, K), lambda b: (0, 0)),
        ],
        out_specs=pl.BlockSpec((1, L, 1), lambda b: (b, 0, 0)),
        out_shape=jax.ShapeDtypeStruct((B, L, 1), jnp.int32),
    )(masked_target_values, project_mat, ct)
    return out[:, :, 0]
